# Initial kernel scaffold; baseline (speedup 1.0000x reference)
import jax
import jax.numpy as jnp
from jax.experimental import pallas as pl


def kernel(feat, edge_index, W1, b1, W2, b2):
    raise NotImplementedError("write your pallas kernel here")


# speedup vs baseline: 9.7860x; 9.7860x over previous
import functools

import jax
import jax.numpy as jnp
from jax import lax
from jax.experimental import pallas as pl
from jax.experimental.pallas import tpu as pltpu
from jax.experimental.pallas import tpu_sc as plsc

N_NODES = 10000
N_EDGES = 320000
NC = 2
NS = 16
NW = NC * NS
EPT = N_EDGES // NW

_MESH = functools.partial(
    plsc.VectorSubcoreMesh, core_axis_name="c", subcore_axis_name="s")


def _degree_kernel():
    CH = 2000

    @functools.partial(
        pl.kernel,
        mesh=_MESH(),
        out_type=(jax.ShapeDtypeStruct((NC * N_NODES,), jnp.float32),
                  jax.ShapeDtypeStruct((NC * N_NODES,), jnp.float32)),
        scratch_types=[
            pltpu.VMEM((CH,), jnp.int32),
            pltpu.VMEM((CH,), jnp.int32),
            pltpu.VMEM((CH,), jnp.float32),
            pltpu.VMEM((1000,), jnp.float32),
            pltpu.VMEM_SHARED((N_NODES,), jnp.float32),
            pltpu.VMEM_SHARED((N_NODES,), jnp.float32),
        ],
    )
    def deg_kernel(src_hbm, dst_hbm, ones_hbm, zeros_hbm, dego_hbm, degi_hbm,
                   src_v, dst_v, ones_v, stage_v, acc_o, acc_i):
        c = lax.axis_index("c")
        s = lax.axis_index("s")
        wid = s * NC + c
        pltpu.sync_copy(ones_hbm, ones_v)

        @pl.when(s < 10)
        def _zero():
            sl = pl.ds(s * 1000, 1000)
            pltpu.sync_copy(zeros_hbm, stage_v)
            pltpu.sync_copy(stage_v, acc_o.at[sl])
            pltpu.sync_copy(stage_v, acc_i.at[sl])

        plsc.subcore_barrier()

        def body(i, carry):
            base = wid * EPT + i * CH
            pltpu.sync_copy(src_hbm.at[pl.ds(base, CH)], src_v)
            pltpu.sync_copy(dst_hbm.at[pl.ds(base, CH)], dst_v)
            pltpu.sync_copy(ones_v, acc_o.at[src_v], add=True)
            pltpu.sync_copy(ones_v, acc_i.at[dst_v], add=True)
            return carry

        lax.fori_loop(0, EPT // CH, body, 0)
        plsc.subcore_barrier()

        @pl.when(s < 10)
        def _write():
            sl = pl.ds(s * 1000, 1000)
            osl = pl.ds(c * N_NODES + s * 1000, 1000)
            pltpu.sync_copy(acc_o.at[sl], stage_v)
            pltpu.sync_copy(stage_v, dego_hbm.at[osl])
            pltpu.sync_copy(acc_i.at[sl], stage_v)
            pltpu.sync_copy(stage_v, degi_hbm.at[osl])

    return deg_kernel


def _aggregate_kernel(D, CH):
    NCHK = N_NODES // CH
    NPASS = (NCHK + NS - 1) // NS

    @functools.partial(
        pl.kernel,
        mesh=_MESH(),
        compiler_params=pltpu.CompilerParams(
            use_tc_tiling_on_sc=(D % 128 == 0)),
        out_type=jax.ShapeDtypeStruct((NC, N_NODES, D), jnp.float32),
        scratch_types=[
            pltpu.VMEM((CH,), jnp.int32),
            pltpu.VMEM((CH,), jnp.int32),
            pltpu.VMEM((CH, D), jnp.float32),
            pltpu.VMEM_SHARED((N_NODES, D), jnp.float32),
            pltpu.SemaphoreType.DMA,
        ],
    )
    def agg_kernel(p_hbm, src_hbm, dst_hbm, zeros_hbm, out_hbm,
                   src_v, dst_v, rows_v, acc_sh, sem):
        c = lax.axis_index("c")
        s = lax.axis_index("s")
        wid = s * NC + c

        pltpu.sync_copy(zeros_hbm, rows_v)
        for j in range(NPASS):
            k = j * NS + s

            @pl.when(k < NCHK)
            def _zero():
                pltpu.sync_copy(rows_v, acc_sh.at[pl.ds(k * CH, CH)])

        plsc.subcore_barrier()

        def body(i, carry):
            base = wid * EPT + i * CH
            pltpu.sync_copy(src_hbm.at[pl.ds(base, CH)], src_v)
            pltpu.async_copy(p_hbm.at[src_v], rows_v, sem).wait()
            pltpu.sync_copy(dst_hbm.at[pl.ds(base, CH)], dst_v)
            pltpu.sync_copy(rows_v, acc_sh.at[dst_v], add=True)
            return carry

        lax.fori_loop(0, EPT // CH, body, 0)
        plsc.subcore_barrier()

        for j in range(NPASS):
            k = j * NS + s

            @pl.when(k < NCHK)
            def _write():
                sl = pl.ds(k * CH, CH)
                pltpu.sync_copy(acc_sh.at[sl], rows_v)
                pltpu.sync_copy(rows_v, out_hbm.at[c, sl])

    return agg_kernel


_B = 1000


def _norm_body(dego_ref, degi_ref, nsrc_ref, ndst_ref):
    do = dego_ref[:, 0:1] + dego_ref[:, 1:2]
    di = degi_ref[:, 0:1] + degi_ref[:, 1:2]
    nsrc_ref[...] = lax.rsqrt(jnp.maximum(do, 1.0))
    ndst_ref[...] = lax.rsqrt(jnp.maximum(di, 1.0))


def _norms(dego_t, degi_t):
    return pl.pallas_call(
        _norm_body,
        grid=(N_NODES // _B,),
        in_specs=[pl.BlockSpec((_B, 2), lambda i: (i, 0)),
                  pl.BlockSpec((_B, 2), lambda i: (i, 0))],
        out_specs=[pl.BlockSpec((_B, 1), lambda i: (i, 0)),
                   pl.BlockSpec((_B, 1), lambda i: (i, 0))],
        out_shape=[jax.ShapeDtypeStruct((N_NODES, 1), jnp.float32),
                   jax.ShapeDtypeStruct((N_NODES, 1), jnp.float32)],
    )(dego_t, degi_t)


def _stage1_body(feat_ref, nsrc_ref, w_ref, out_ref):
    h = feat_ref[...] * nsrc_ref[...]
    out_ref[...] = jnp.dot(h, w_ref[...], preferred_element_type=jnp.float32)


def _stage1(feat, nsrc, W1):
    return pl.pallas_call(
        _stage1_body,
        grid=(N_NODES // _B,),
        in_specs=[pl.BlockSpec((_B, 128), lambda i: (i, 0)),
                  pl.BlockSpec((_B, 1), lambda i: (i, 0)),
                  pl.BlockSpec((128, 128), lambda i: (0, 0))],
        out_specs=pl.BlockSpec((_B, 128), lambda i: (i, 0)),
        out_shape=jax.ShapeDtypeStruct((N_NODES, 128), jnp.float32),
    )(feat, nsrc, W1)


def _stage2_body(agg_ref, ndst_ref, b1_ref, nsrc_ref, w2_ref, out_ref):
    agg = agg_ref[0] + agg_ref[1]
    h = jnp.maximum(agg * ndst_ref[...] + b1_ref[...], 0.0)
    h = h * nsrc_ref[...]
    out_ref[...] = jnp.dot(h, w2_ref[...], preferred_element_type=jnp.float32)


def _stage2(agg1, ndst, b1, nsrc, W2p, DP):
    return pl.pallas_call(
        _stage2_body,
        grid=(N_NODES // _B,),
        in_specs=[pl.BlockSpec((NC, _B, 128), lambda i: (0, i, 0)),
                  pl.BlockSpec((_B, 1), lambda i: (i, 0)),
                  pl.BlockSpec((1, 128), lambda i: (0, 0)),
                  pl.BlockSpec((_B, 1), lambda i: (i, 0)),
                  pl.BlockSpec((128, DP), lambda i: (0, 0))],
        out_specs=pl.BlockSpec((_B, DP), lambda i: (i, 0)),
        out_shape=jax.ShapeDtypeStruct((N_NODES, DP), jnp.float32),
    )(agg1, ndst, b1, nsrc, W2p)


def _stage3_body(agg_ref, ndst_ref, b2_ref, out_ref):
    agg = agg_ref[0] + agg_ref[1]
    out_ref[...] = agg * ndst_ref[...] + b2_ref[...]


def _stage3(agg2, ndst, b2p, DP):
    return pl.pallas_call(
        _stage3_body,
        grid=(N_NODES // _B,),
        in_specs=[pl.BlockSpec((NC, _B, DP), lambda i: (0, i, 0)),
                  pl.BlockSpec((_B, 1), lambda i: (i, 0)),
                  pl.BlockSpec((1, DP), lambda i: (0, 0))],
        out_specs=pl.BlockSpec((_B, DP), lambda i: (i, 0)),
        out_shape=jax.ShapeDtypeStruct((N_NODES, DP), jnp.float32),
    )(agg2, ndst, b2p)


DP2 = 48


def kernel(feat, edge_index, W1, b1, W2, b2):
    src = edge_index[0].astype(jnp.int32)
    dst = edge_index[1].astype(jnp.int32)

    ones_e = jnp.ones((2000,), jnp.float32)
    zeros_n = jnp.zeros((1000,), jnp.float32)
    zeros_128 = jnp.zeros((200, 128), jnp.float32)
    zeros_dp = jnp.zeros((1000, DP2), jnp.float32)

    dego_p, degi_p = _degree_kernel()(src, dst, ones_e, zeros_n)
    nsrc, ndst = _norms(dego_p.reshape(NC, N_NODES).T,
                        degi_p.reshape(NC, N_NODES).T)

    p1 = _stage1(feat, nsrc, W1)
    agg1 = _aggregate_kernel(128, 200)(p1, src, dst, zeros_128)

    W2p = jnp.pad(W2, ((0, 0), (0, DP2 - W2.shape[1])))
    b2p = jnp.pad(b2, (0, DP2 - b2.shape[0]))
    p2 = _stage2(agg1, ndst, b1.reshape(1, 128), nsrc, W2p, DP2)
    agg2 = _aggregate_kernel(DP2, 1000)(p2, src, dst, zeros_dp)

    out = _stage3(agg2, ndst, b2p.reshape(1, DP2), DP2)
    return out[:, :W2.shape[1]]


# speedup vs baseline: 13.5697x; 1.3866x over previous
import functools

import jax
import jax.numpy as jnp
from jax import lax
from jax.experimental import pallas as pl
from jax.experimental.pallas import tpu as pltpu
from jax.experimental.pallas import tpu_sc as plsc

N_NODES = 10000
N_EDGES = 320000
NC = 2
NS = 16
NW = NC * NS
EPT = N_EDGES // NW

_MESH = functools.partial(
    plsc.VectorSubcoreMesh, core_axis_name="c", subcore_axis_name="s")


def _degree_kernel():
    CH = 2000

    @functools.partial(
        pl.kernel,
        mesh=_MESH(),
        out_type=(jax.ShapeDtypeStruct((NC * N_NODES,), jnp.float32),
                  jax.ShapeDtypeStruct((NC * N_NODES,), jnp.float32)),
        scratch_types=[
            pltpu.VMEM((CH,), jnp.int32),
            pltpu.VMEM((CH,), jnp.int32),
            pltpu.VMEM((CH,), jnp.float32),
            pltpu.VMEM((1000,), jnp.float32),
            pltpu.VMEM_SHARED((N_NODES,), jnp.float32),
            pltpu.VMEM_SHARED((N_NODES,), jnp.float32),
        ],
    )
    def deg_kernel(src_hbm, dst_hbm, ones_hbm, zeros_hbm, dego_hbm, degi_hbm,
                   src_v, dst_v, ones_v, stage_v, acc_o, acc_i):
        c = lax.axis_index("c")
        s = lax.axis_index("s")
        wid = s * NC + c
        pltpu.sync_copy(ones_hbm, ones_v)

        @pl.when(s < 10)
        def _zero():
            sl = pl.ds(s * 1000, 1000)
            pltpu.sync_copy(zeros_hbm, stage_v)
            pltpu.sync_copy(stage_v, acc_o.at[sl])
            pltpu.sync_copy(stage_v, acc_i.at[sl])

        plsc.subcore_barrier()

        def body(i, carry):
            base = wid * EPT + i * CH
            pltpu.sync_copy(src_hbm.at[pl.ds(base, CH)], src_v)
            pltpu.sync_copy(dst_hbm.at[pl.ds(base, CH)], dst_v)
            pltpu.sync_copy(ones_v, acc_o.at[src_v], add=True)
            pltpu.sync_copy(ones_v, acc_i.at[dst_v], add=True)
            return carry

        lax.fori_loop(0, EPT // CH, body, 0)
        plsc.subcore_barrier()

        @pl.when(s < 10)
        def _write():
            sl = pl.ds(s * 1000, 1000)
            osl = pl.ds(c * N_NODES + s * 1000, 1000)
            pltpu.sync_copy(acc_o.at[sl], stage_v)
            pltpu.sync_copy(stage_v, dego_hbm.at[osl])
            pltpu.sync_copy(acc_i.at[sl], stage_v)
            pltpu.sync_copy(stage_v, degi_hbm.at[osl])

    return deg_kernel


def _aggregate_kernel(D, CH):
    NCH = EPT // CH
    NZCHK = N_NODES // CH
    NPASS = (NZCHK + NS - 1) // NS

    @functools.partial(
        pl.kernel,
        mesh=_MESH(),
        compiler_params=pltpu.CompilerParams(
            use_tc_tiling_on_sc=(D % 128 == 0)),
        out_type=jax.ShapeDtypeStruct((NC, N_NODES, D), jnp.float32),
        scratch_types=[
            pltpu.VMEM((EPT,), jnp.int32),
            pltpu.VMEM((NCH, CH), jnp.int32),
            pltpu.VMEM((CH, D), jnp.float32),
            pltpu.VMEM((CH, D), jnp.float32),
            pltpu.VMEM_SHARED((N_NODES, D), jnp.float32),
            pltpu.SemaphoreType.DMA,
            pltpu.SemaphoreType.DMA,
        ],
    )
    def agg_kernel(p_hbm, src_hbm, dst_hbm, zeros_hbm, out_hbm,
                   src_v, dst_v, rows_v0, rows_v1, acc_sh, sem0, sem1):
        c = lax.axis_index("c")
        s = lax.axis_index("s")
        wid = s * NC + c
        rows = (rows_v0, rows_v1)
        sems = (sem0, sem1)

        pltpu.sync_copy(zeros_hbm, rows_v0)
        for j in range(NPASS):
            k = j * NS + s

            @pl.when(k < NZCHK)
            def _zero():
                pltpu.sync_copy(rows_v0, acc_sh.at[pl.ds(k * CH, CH)])

        pltpu.sync_copy(src_hbm.at[wid], src_v)
        pltpu.sync_copy(dst_hbm.at[wid], dst_v)
        plsc.subcore_barrier()

        def _gather(i, b):
            pltpu.async_copy(p_hbm.at[src_v.at[pl.ds(i * CH, CH)]],
                             rows[b], sems[b])

        def _gather_wait(i, b):
            pltpu.make_async_copy(p_hbm.at[src_v.at[pl.ds(i * CH, CH)]],
                                  rows[b], sems[b]).wait()

        _gather(0, 0)
        _gather(1, 1)

        def pair(g, carry):
            for b in range(2):
                i = 2 * g + b
                _gather_wait(i, b)
                pltpu.sync_copy(rows[b], acc_sh.at[dst_v.at[i]], add=True)
                nxt = i + 2

                @pl.when(nxt < NCH)
                def _next():
                    _gather(nxt, b)
            return carry

        lax.fori_loop(0, NCH // 2, pair, 0)
        if NCH % 2:
            i = NCH - 1
            _gather_wait(i, 0)
            pltpu.sync_copy(rows[0], acc_sh.at[dst_v.at[i]], add=True)

        plsc.subcore_barrier()

        for j in range(NPASS):
            k = j * NS + s

            @pl.when(k < NZCHK)
            def _write():
                sl = pl.ds(k * CH, CH)
                pltpu.sync_copy(acc_sh.at[sl], rows_v0)
                pltpu.sync_copy(rows_v0, out_hbm.at[c, sl])

    return agg_kernel


_B = 1000


def _norm_body(dego_ref, degi_ref, nsrc_ref, ndst_ref):
    do = dego_ref[:, 0:1] + dego_ref[:, 1:2]
    di = degi_ref[:, 0:1] + degi_ref[:, 1:2]
    nsrc_ref[...] = lax.rsqrt(jnp.maximum(do, 1.0))
    ndst_ref[...] = lax.rsqrt(jnp.maximum(di, 1.0))


def _norms(dego_t, degi_t):
    return pl.pallas_call(
        _norm_body,
        grid=(N_NODES // _B,),
        in_specs=[pl.BlockSpec((_B, 2), lambda i: (i, 0)),
                  pl.BlockSpec((_B, 2), lambda i: (i, 0))],
        out_specs=[pl.BlockSpec((_B, 1), lambda i: (i, 0)),
                   pl.BlockSpec((_B, 1), lambda i: (i, 0))],
        out_shape=[jax.ShapeDtypeStruct((N_NODES, 1), jnp.float32),
                   jax.ShapeDtypeStruct((N_NODES, 1), jnp.float32)],
    )(dego_t, degi_t)


def _stage1_body(feat_ref, nsrc_ref, w_ref, out_ref):
    h = feat_ref[...] * nsrc_ref[...]
    out_ref[...] = jnp.dot(h, w_ref[...], preferred_element_type=jnp.float32)


def _stage1(feat, nsrc, W1):
    return pl.pallas_call(
        _stage1_body,
        grid=(N_NODES // _B,),
        in_specs=[pl.BlockSpec((_B, 128), lambda i: (i, 0)),
                  pl.BlockSpec((_B, 1), lambda i: (i, 0)),
                  pl.BlockSpec((128, 128), lambda i: (0, 0))],
        out_specs=pl.BlockSpec((_B, 128), lambda i: (i, 0)),
        out_shape=jax.ShapeDtypeStruct((N_NODES, 128), jnp.float32),
    )(feat, nsrc, W1)


def _stage2_body(agg_ref, ndst_ref, b1_ref, nsrc_ref, w2_ref, out_ref):
    agg = agg_ref[0] + agg_ref[1]
    h = jnp.maximum(agg * ndst_ref[...] + b1_ref[...], 0.0)
    h = h * nsrc_ref[...]
    out_ref[...] = jnp.dot(h, w2_ref[...], preferred_element_type=jnp.float32)


def _stage2(agg1, ndst, b1, nsrc, W2p, DP):
    return pl.pallas_call(
        _stage2_body,
        grid=(N_NODES // _B,),
        in_specs=[pl.BlockSpec((NC, _B, 128), lambda i: (0, i, 0)),
                  pl.BlockSpec((_B, 1), lambda i: (i, 0)),
                  pl.BlockSpec((1, 128), lambda i: (0, 0)),
                  pl.BlockSpec((_B, 1), lambda i: (i, 0)),
                  pl.BlockSpec((128, DP), lambda i: (0, 0))],
        out_specs=pl.BlockSpec((_B, DP), lambda i: (i, 0)),
        out_shape=jax.ShapeDtypeStruct((N_NODES, DP), jnp.float32),
    )(agg1, ndst, b1, nsrc, W2p)


def _stage3_body(agg_ref, ndst_ref, b2_ref, out_ref):
    agg = agg_ref[0] + agg_ref[1]
    out_ref[...] = agg * ndst_ref[...] + b2_ref[...]


def _stage3(agg2, ndst, b2p, DP):
    return pl.pallas_call(
        _stage3_body,
        grid=(N_NODES // _B,),
        in_specs=[pl.BlockSpec((NC, _B, DP), lambda i: (0, i, 0)),
                  pl.BlockSpec((_B, 1), lambda i: (i, 0)),
                  pl.BlockSpec((1, DP), lambda i: (0, 0))],
        out_specs=pl.BlockSpec((_B, DP), lambda i: (i, 0)),
        out_shape=jax.ShapeDtypeStruct((N_NODES, DP), jnp.float32),
    )(agg2, ndst, b2p)


DP2 = 48


def kernel(feat, edge_index, W1, b1, W2, b2):
    src = edge_index[0].astype(jnp.int32)
    dst = edge_index[1].astype(jnp.int32)

    ones_e = jnp.ones((2000,), jnp.float32)
    zeros_n = jnp.zeros((1000,), jnp.float32)
    zeros_128 = jnp.zeros((80, 128), jnp.float32)
    zeros_dp = jnp.zeros((400, DP2), jnp.float32)

    dego_p, degi_p = _degree_kernel()(src, dst, ones_e, zeros_n)
    nsrc, ndst = _norms(dego_p.reshape(NC, N_NODES).T,
                        degi_p.reshape(NC, N_NODES).T)

    p1 = _stage1(feat, nsrc, W1)
    src_t = src.reshape(NW, EPT)
    dst1 = dst.reshape(NW, EPT // 80, 80)
    agg1 = _aggregate_kernel(128, 80)(p1, src_t, dst1, zeros_128)

    W2p = jnp.pad(W2, ((0, 0), (0, DP2 - W2.shape[1])))
    b2p = jnp.pad(b2, (0, DP2 - b2.shape[0]))
    p2 = _stage2(agg1, ndst, b1.reshape(1, 128), nsrc, W2p, DP2)
    dst2 = dst.reshape(NW, EPT // 400, 400)
    agg2 = _aggregate_kernel(DP2, 400)(p2, src_t, dst2, zeros_dp)

    out = _stage3(agg2, ndst, b2p.reshape(1, DP2), DP2)
    return out[:, :W2.shape[1]]


# speedup vs baseline: 15.6114x; 1.1505x over previous
import functools

import jax
import jax.numpy as jnp
from jax import lax
from jax.experimental import pallas as pl
from jax.experimental.pallas import tpu as pltpu
from jax.experimental.pallas import tpu_sc as plsc

N_NODES = 10000
N_EDGES = 320000
NC = 2
NS = 16
NW = NC * NS
EPT = N_EDGES // NW

_MESH = functools.partial(
    plsc.VectorSubcoreMesh, core_axis_name="c", subcore_axis_name="s")


def _degree_kernel():
    @functools.partial(
        pl.kernel,
        mesh=_MESH(),
        out_type=(jax.ShapeDtypeStruct((NC * N_NODES,), jnp.float32),
                  jax.ShapeDtypeStruct((NC * N_NODES,), jnp.float32)),
        scratch_types=[
            pltpu.VMEM((EPT,), jnp.int32),
            pltpu.VMEM((EPT,), jnp.int32),
            pltpu.VMEM((EPT,), jnp.float32),
            pltpu.VMEM((1000,), jnp.float32),
            pltpu.VMEM_SHARED((N_NODES,), jnp.float32),
            pltpu.VMEM_SHARED((N_NODES,), jnp.float32),
        ],
    )
    def deg_kernel(src_hbm, dst_hbm, ones_hbm, zeros_hbm, dego_hbm, degi_hbm,
                   src_v, dst_v, ones_v, stage_v, acc_o, acc_i):
        c = lax.axis_index("c")
        s = lax.axis_index("s")
        wid = s * NC + c
        pltpu.sync_copy(ones_hbm, ones_v)
        pltpu.sync_copy(src_hbm.at[wid], src_v)
        pltpu.sync_copy(dst_hbm.at[wid], dst_v)

        @pl.when(s < 10)
        def _zero():
            sl = pl.ds(s * 1000, 1000)
            pltpu.sync_copy(zeros_hbm, stage_v)
            pltpu.sync_copy(stage_v, acc_o.at[sl])
            pltpu.sync_copy(stage_v, acc_i.at[sl])

        plsc.subcore_barrier()
        pltpu.sync_copy(ones_v, acc_o.at[src_v], add=True)
        pltpu.sync_copy(ones_v, acc_i.at[dst_v], add=True)
        plsc.subcore_barrier()

        @pl.when(s < 10)
        def _write():
            sl = pl.ds(s * 1000, 1000)
            osl = pl.ds(c * N_NODES + s * 1000, 1000)
            pltpu.sync_copy(acc_o.at[sl], stage_v)
            pltpu.sync_copy(stage_v, dego_hbm.at[osl])
            pltpu.sync_copy(acc_i.at[sl], stage_v)
            pltpu.sync_copy(stage_v, degi_hbm.at[osl])

    return deg_kernel


def _aggregate_kernel(D, CH, NBUF):
    NCH = EPT // CH
    NZCHK = N_NODES // CH
    NPASS = (NZCHK + NS - 1) // NS

    @functools.partial(
        pl.kernel,
        mesh=_MESH(),
        compiler_params=pltpu.CompilerParams(
            use_tc_tiling_on_sc=(D % 128 == 0)),
        out_type=jax.ShapeDtypeStruct((NC, N_NODES, D), jnp.float32),
        scratch_types=[
            pltpu.VMEM((EPT,), jnp.int32),
            pltpu.VMEM((EPT,), jnp.int32),
        ] + [pltpu.VMEM((CH, D), jnp.float32) for _ in range(NBUF)]
          + [pltpu.VMEM_SHARED((N_NODES, D), jnp.float32)]
          + [pltpu.SemaphoreType.DMA for _ in range(NBUF)],
    )
    def agg_kernel(p_hbm, src_hbm, dst_hbm, zeros_hbm, out_hbm, *scratch):
        src_v, dst_v = scratch[0], scratch[1]
        rows = scratch[2:2 + NBUF]
        acc_sh = scratch[2 + NBUF]
        sems = scratch[3 + NBUF:3 + 2 * NBUF]
        c = lax.axis_index("c")
        s = lax.axis_index("s")
        wid = s * NC + c

        pltpu.sync_copy(zeros_hbm, rows[0])
        for j in range(NPASS):
            k = j * NS + s

            @pl.when(k < NZCHK)
            def _zero():
                pltpu.sync_copy(rows[0], acc_sh.at[pl.ds(k * CH, CH)])

        pltpu.sync_copy(src_hbm.at[wid], src_v)
        pltpu.sync_copy(dst_hbm.at[wid], dst_v)
        plsc.subcore_barrier()

        def _gather(i, b):
            pltpu.async_copy(p_hbm.at[src_v.at[pl.ds(i * CH, CH)]],
                             rows[b], sems[b])

        def _gather_wait(i, b):
            pltpu.make_async_copy(p_hbm.at[src_v.at[pl.ds(i * CH, CH)]],
                                  rows[b], sems[b]).wait()

        def _scatter(i, b):
            pltpu.sync_copy(rows[b], acc_sh.at[dst_v.at[pl.ds(i * CH, CH)]],
                            add=True)

        for b in range(NBUF):
            _gather(b, b)

        def group(g, carry):
            for b in range(NBUF):
                i = g * NBUF + b
                _gather_wait(i, b)
                _scatter(i, b)
                nxt = i + NBUF

                @pl.when(nxt < NCH)
                def _next():
                    _gather(nxt, b)
            return carry

        lax.fori_loop(0, NCH // NBUF, group, 0)
        for b in range(NCH % NBUF):
            i = (NCH // NBUF) * NBUF + b
            _gather_wait(i, b)
            _scatter(i, b)

        plsc.subcore_barrier()

        for j in range(NPASS):
            k = j * NS + s

            @pl.when(k < NZCHK)
            def _write():
                sl = pl.ds(k * CH, CH)
                pltpu.sync_copy(acc_sh.at[sl], rows[0])
                pltpu.sync_copy(rows[0], out_hbm.at[c, sl])

    return agg_kernel


_B = 1000


def _nrm(deg_ref):
    d = deg_ref[:, 0:1] + deg_ref[:, 1:2]
    return lax.rsqrt(jnp.maximum(d, 1.0))


def _stage1_body(feat_ref, dego_ref, w_ref, out_ref):
    h = feat_ref[...] * _nrm(dego_ref)
    out_ref[...] = jnp.dot(h, w_ref[...], preferred_element_type=jnp.float32)


def _stage1(feat, dego_t, W1):
    return pl.pallas_call(
        _stage1_body,
        grid=(N_NODES // _B,),
        in_specs=[pl.BlockSpec((_B, 128), lambda i: (i, 0)),
                  pl.BlockSpec((_B, 2), lambda i: (i, 0)),
                  pl.BlockSpec((128, 128), lambda i: (0, 0))],
        out_specs=pl.BlockSpec((_B, 128), lambda i: (i, 0)),
        out_shape=jax.ShapeDtypeStruct((N_NODES, 128), jnp.float32),
    )(feat, dego_t, W1)


def _stage2_body(agg_ref, dego_ref, degi_ref, b1_ref, w2_ref, out_ref):
    agg = agg_ref[0] + agg_ref[1]
    h = jnp.maximum(agg * _nrm(degi_ref) + b1_ref[...], 0.0)
    h = h * _nrm(dego_ref)
    out_ref[...] = jnp.dot(h, w2_ref[...], preferred_element_type=jnp.float32)


def _stage2(agg1, dego_t, degi_t, b1, W2p, DP):
    return pl.pallas_call(
        _stage2_body,
        grid=(N_NODES // _B,),
        in_specs=[pl.BlockSpec((NC, _B, 128), lambda i: (0, i, 0)),
                  pl.BlockSpec((_B, 2), lambda i: (i, 0)),
                  pl.BlockSpec((_B, 2), lambda i: (i, 0)),
                  pl.BlockSpec((1, 128), lambda i: (0, 0)),
                  pl.BlockSpec((128, DP), lambda i: (0, 0))],
        out_specs=pl.BlockSpec((_B, DP), lambda i: (i, 0)),
        out_shape=jax.ShapeDtypeStruct((N_NODES, DP), jnp.float32),
    )(agg1, dego_t, degi_t, b1, W2p)


def _stage3_body(agg_ref, degi_ref, b2_ref, out_ref):
    agg = agg_ref[0] + agg_ref[1]
    out_ref[...] = agg * _nrm(degi_ref) + b2_ref[...]


def _stage3(agg2, degi_t, b2p, DP):
    return pl.pallas_call(
        _stage3_body,
        grid=(N_NODES // _B,),
        in_specs=[pl.BlockSpec((NC, _B, DP), lambda i: (0, i, 0)),
                  pl.BlockSpec((_B, 2), lambda i: (i, 0)),
                  pl.BlockSpec((1, DP), lambda i: (0, 0))],
        out_specs=pl.BlockSpec((_B, DP), lambda i: (i, 0)),
        out_shape=jax.ShapeDtypeStruct((N_NODES, DP), jnp.float32),
    )(agg2, degi_t, b2p)


DP2 = 48


def kernel(feat, edge_index, W1, b1, W2, b2):
    src = edge_index[0].astype(jnp.int32)
    dst = edge_index[1].astype(jnp.int32)
    src_t = src.reshape(NW, EPT)
    dst_t = dst.reshape(NW, EPT)

    ones_e = jnp.ones((EPT,), jnp.float32)
    zeros_n = jnp.zeros((1000,), jnp.float32)
    zeros_128 = jnp.zeros((80, 128), jnp.float32)
    zeros_dp = jnp.zeros((400, DP2), jnp.float32)

    dego_p, degi_p = _degree_kernel()(src_t, dst_t, ones_e, zeros_n)
    dego_t = dego_p.reshape(NC, N_NODES).T
    degi_t = degi_p.reshape(NC, N_NODES).T

    p1 = _stage1(feat, dego_t, W1)
    agg1 = _aggregate_kernel(128, 80, 3)(p1, src_t, dst_t, zeros_128)

    W2p = jnp.pad(W2, ((0, 0), (0, DP2 - W2.shape[1])))
    b2p = jnp.pad(b2, (0, DP2 - b2.shape[0]))
    p2 = _stage2(agg1, dego_t, degi_t, b1.reshape(1, 128), W2p, DP2)
    agg2 = _aggregate_kernel(DP2, 400, 3)(p2, src_t, dst_t, zeros_dp)

    out = _stage3(agg2, degi_t, b2p.reshape(1, DP2), DP2)
    return out[:, :W2.shape[1]]
